# Initial kernel scaffold; baseline (speedup 1.0000x reference)
import jax
import jax.numpy as jnp
from jax.experimental import pallas as pl


def kernel(x, gate_w, lattice_weights, out_w, out_b, ln_g, ln_b):
    raise NotImplementedError("write your pallas kernel here")


# speedup vs baseline: 34.5897x; 34.5897x over previous
import jax
import jax.numpy as jnp
from jax.experimental import pallas as pl
from jax.experimental.pallas import tpu as pltpu

_B, _S, _D, _L = 1, 512, 768, 16


def _hyper_lattice_kernel(x_ref, gate_w_ref, w_ref, out_w_ref, out_b_ref,
                          ln_g_ref, ln_b_ref, out_ref, acc_ref, idx_ref):
    e = pl.program_id(0)
    x = x_ref[...]

    @pl.when(e == 0)
    def _route():
        logits = jnp.dot(x, gate_w_ref[...].T,
                         preferred_element_type=jnp.float32)
        idx_ref[...] = jnp.argmax(logits, axis=-1, keepdims=True).astype(
            jnp.int32)

    mask = idx_ref[...] == e
    xm = jnp.where(mask, x, 0.0)
    contrib = jnp.dot(xm, w_ref[0], preferred_element_type=jnp.float32)

    @pl.when(e == 0)
    def _first():
        acc_ref[...] = contrib

    @pl.when(e > 0)
    def _accum():
        acc_ref[...] += contrib

    @pl.when(e == _L - 1)
    def _epilogue():
        out2 = jnp.dot(acc_ref[...], out_w_ref[...].T,
                       preferred_element_type=jnp.float32) + out_b_ref[...]
        h = x + out2
        mu = jnp.mean(h, axis=-1, keepdims=True)
        var = jnp.mean((h - mu) ** 2, axis=-1, keepdims=True)
        out_ref[...] = ((h - mu) * jax.lax.rsqrt(var + 1e-5)
                        * ln_g_ref[...] + ln_b_ref[...])


def kernel(x, gate_w, lattice_weights, out_w, out_b, ln_g, ln_b):
    x2 = x.reshape(_S, _D)
    out = pl.pallas_call(
        _hyper_lattice_kernel,
        grid=(_L,),
        in_specs=[
            pl.BlockSpec((_S, _D), lambda e: (0, 0)),
            pl.BlockSpec((_L, _D), lambda e: (0, 0)),
            pl.BlockSpec((1, _D, _D), lambda e: (e, 0, 0)),
            pl.BlockSpec((_D, _D), lambda e: (0, 0)),
            pl.BlockSpec((1, _D), lambda e: (0, 0)),
            pl.BlockSpec((1, _D), lambda e: (0, 0)),
            pl.BlockSpec((1, _D), lambda e: (0, 0)),
        ],
        out_specs=pl.BlockSpec((_S, _D), lambda e: (0, 0)),
        out_shape=jax.ShapeDtypeStruct((_S, _D), jnp.float32),
        scratch_shapes=[
            pltpu.VMEM((_S, _D), jnp.float32),
            pltpu.VMEM((_S, 1), jnp.int32),
        ],
    )(x2, gate_w, lattice_weights, out_w,
      out_b.reshape(1, _D), ln_g.reshape(1, _D), ln_b.reshape(1, _D))
    return out.reshape(_B, _S, _D)
